# Initial kernel scaffold; baseline (speedup 1.0000x reference)
import jax
import jax.numpy as jnp
from jax.experimental import pallas as pl


def kernel(x, edge_index, edge_weight, W1, bias, Wfc, bfc):
    raise NotImplementedError("write your pallas kernel here")


# speedup vs baseline: 70.2108x; 70.2108x over previous
import functools

import jax
import jax.numpy as jnp
from jax import lax
from jax.experimental import pallas as pl
from jax.experimental.pallas import tpu as pltpu
from jax.experimental.pallas import tpu_sc as plsc

N = 10000
E = 160000
D = 256
H = 256
O = 256

NC = 2
NS = 16
L = 16
NW = NC * NS

EC_MIN = E // NS
EC_H = E // NW
MIN_VREGS = EC_MIN // L
H_VREGS = (EC_H + L - 1) // L

_INT32_MAX = 2147483647


def _sc_body(row_hbm, col_hbm, deg_out, cnt_out, idx_out,
             row_v, col_v, deg_v, cnt_v, stage_v, shared_mins, allmin_v):
    c_idx = lax.axis_index("c")
    s_idx = lax.axis_index("s")
    wid = s_idx * NC + c_idx

    iota = lax.iota(jnp.int32, L)
    ones = jnp.full((L,), 1.0, jnp.float32)
    zeros = jnp.zeros((L,), jnp.float32)

    pltpu.sync_copy(row_hbm.at[pl.ds(s_idx * EC_MIN, EC_MIN)],
                    row_v.at[pl.ds(0, EC_MIN)])
    pltpu.sync_copy(col_hbm.at[pl.ds(s_idx * EC_MIN + c_idx * EC_H, EC_H)],
                    col_v.at[pl.ds(0, EC_H)])

    def zero_body(i, carry):
        deg_v[pl.ds(i * L, L)] = zeros
        cnt_v[pl.ds(i * L, L)] = zeros
        return carry
    lax.fori_loop(0, N // L, zero_body, 0)

    def min_body(i, acc):
        return jnp.minimum(acc, row_v[pl.ds(i * L, L)])
    acc = lax.fori_loop(0, MIN_VREGS, min_body,
                        jnp.full((L,), _INT32_MAX, jnp.int32))

    def deg_body(i, carry):
        off = i * L
        cvals = col_v[pl.ds(off, L)]
        m = (off + iota) < EC_H
        plsc.addupdate_scatter(deg_v, [cvals], ones, mask=m)
        return carry
    lax.fori_loop(0, H_VREGS, deg_body, 0)

    stage_v[...] = acc
    pltpu.sync_copy(stage_v, shared_mins.at[pl.ds(s_idx * L, L)])
    plsc.subcore_barrier()
    pltpu.sync_copy(shared_mins, allmin_v)
    macc = allmin_v[pl.ds(0, L)]
    for t in range(1, NS):
        macc = jnp.minimum(macc, allmin_v[pl.ds(t * L, L)])
    gmin = jnp.min(macc)

    def cnt_body(i, carry):
        off = i * L
        rvals = row_v[pl.ds(c_idx * EC_H + off, L)]
        cvals = col_v[pl.ds(off, L)]
        m = ((off + iota) < EC_H) & (cvals == gmin)
        plsc.addupdate_scatter(cnt_v, [rvals], ones, mask=m)
        return carry
    lax.fori_loop(0, H_VREGS, cnt_body, 0)

    pltpu.sync_copy(deg_v, deg_out.at[wid])
    pltpu.sync_copy(cnt_v, cnt_out.at[wid])

    @pl.when(wid == 0)
    def _():
        stage_v[...] = jnp.full((L,), 0, jnp.int32) + gmin
        pltpu.sync_copy(stage_v, idx_out)


@jax.jit
def _sc_hist(row, col):
    kern = pl.kernel(
        _sc_body,
        out_type=(
            jax.ShapeDtypeStruct((NW, N), jnp.float32),
            jax.ShapeDtypeStruct((NW, N), jnp.float32),
            jax.ShapeDtypeStruct((L,), jnp.int32),
        ),
        mesh=plsc.VectorSubcoreMesh(core_axis_name="c", subcore_axis_name="s"),
        compiler_params=pltpu.CompilerParams(needs_layout_passes=False),
        scratch_types=[
            pltpu.VMEM((EC_MIN + L, ), jnp.int32),
            pltpu.VMEM((EC_H + L,), jnp.int32),
            pltpu.VMEM((N,), jnp.float32),
            pltpu.VMEM((N,), jnp.float32),
            pltpu.VMEM((L,), jnp.int32),
            pltpu.VMEM_SHARED((NS * L,), jnp.int32),
            pltpu.VMEM((NS * L,), jnp.int32),
        ],
    )
    return kern(row, col)


def _tc_body(idx_ref, degp_ref, cntp_ref, x_ref, w1_ref, wfc_ref,
             bias_ref, bfc_ref, y_ref):
    deg = jnp.sum(degp_ref[...], axis=0, keepdims=True)
    cnt = jnp.sum(cntp_ref[...], axis=0, keepdims=True)
    w = cnt * jnp.sqrt(deg)
    idx = idx_ref[0]
    onehot = (lax.broadcasted_iota(jnp.int32, (1, N), 1) == idx)
    scale = jnp.sqrt(jnp.sum(jnp.where(onehot, deg, 0.0)))
    s = lax.dot_general(w, x_ref[...], (((1,), (0,)), ((), ())),
                        preferred_element_type=jnp.float32,
                        precision=lax.Precision.HIGHEST)
    z = lax.dot_general(s, w1_ref[...], (((1,), (1,)), ((), ())),
                        preferred_element_type=jnp.float32,
                        precision=lax.Precision.HIGHEST)
    out_row = scale * z + bias_ref[...]
    y = lax.dot_general(out_row, wfc_ref[...], (((1,), (1,)), ((), ())),
                        preferred_element_type=jnp.float32,
                        precision=lax.Precision.HIGHEST) + bfc_ref[...]
    y_ref[...] = y


@jax.jit
def _tc_finish(idx1, degp, cntp, x, W1, Wfc, bias2, bfc2):
    return pl.pallas_call(
        _tc_body,
        out_shape=jax.ShapeDtypeStruct((1, O), jnp.float32),
        in_specs=[
            pl.BlockSpec(memory_space=pltpu.SMEM),
            pl.BlockSpec(memory_space=pltpu.VMEM),
            pl.BlockSpec(memory_space=pltpu.VMEM),
            pl.BlockSpec(memory_space=pltpu.VMEM),
            pl.BlockSpec(memory_space=pltpu.VMEM),
            pl.BlockSpec(memory_space=pltpu.VMEM),
            pl.BlockSpec(memory_space=pltpu.VMEM),
            pl.BlockSpec(memory_space=pltpu.VMEM),
        ],
        out_specs=pl.BlockSpec(memory_space=pltpu.VMEM),
    )(idx1, degp, cntp, x, W1, Wfc, bias2, bfc2)


def kernel(x, edge_index, edge_weight, W1, bias, Wfc, bfc):
    row = edge_index[0].astype(jnp.int32)
    col = edge_index[1].astype(jnp.int32)
    degp, cntp, idxv = _sc_hist(row, col)
    y = _tc_finish(idxv[:1], degp, cntp, x, W1, Wfc,
                   bias.reshape(1, H), bfc.reshape(1, O))
    return y.reshape(O)


# speedup vs baseline: 87.4653x; 1.2458x over previous
import jax
import jax.numpy as jnp
from jax import lax
from jax.experimental import pallas as pl
from jax.experimental.pallas import tpu as pltpu
from jax.experimental.pallas import tpu_sc as plsc

N = 10000
E = 160000
D = 256
H = 256
O = 256

NC = 2
NS = 16
L = 16
NW = NC * NS

EC_MIN = E // NS
EC_H = E // NW
U = 4
ZU = 5
H_FULL = EC_H // (L * U)
H_TAIL = EC_H - H_FULL * L * U

_INT32_MAX = 2147483647


def _sc_body(ei_hbm, deg_out, cnt_out, idx_out,
             row_v, col_v, deg_v, cnt_v, stage_v, shared_mins, allmin_v):
    c_idx = lax.axis_index("c")
    s_idx = lax.axis_index("s")
    wid = s_idx * NC + c_idx

    iota = lax.iota(jnp.int32, L)
    ones = jnp.full((L,), 1.0, jnp.float32)
    zeros = jnp.zeros((L,), jnp.float32)

    pltpu.sync_copy(ei_hbm.at[pl.ds(s_idx * EC_MIN, EC_MIN)],
                    row_v.at[pl.ds(0, EC_MIN)])
    pltpu.sync_copy(ei_hbm.at[pl.ds(E + s_idx * EC_MIN + c_idx * EC_H, EC_H)],
                    col_v.at[pl.ds(0, EC_H)])

    def zm_body(i, acc):
        base = i * (L * ZU)
        for u in range(ZU):
            off = base + u * L
            deg_v[pl.ds(off, L)] = zeros
            cnt_v[pl.ds(off, L)] = zeros
            acc = jnp.minimum(acc, row_v[pl.ds(off, L)])
        return acc
    acc = lax.fori_loop(0, (N // L) // ZU, zm_body,
                        jnp.full((L,), _INT32_MAX, jnp.int32))

    def deg_body(i, carry):
        base = i * (L * U)
        for u in range(U):
            off = base + u * L
            plsc.addupdate_scatter(deg_v, [col_v[pl.ds(off, L)]], ones)
        return carry
    lax.fori_loop(0, H_FULL, deg_body, 0)
    tail_off = H_FULL * L * U
    m_tail = iota < H_TAIL
    plsc.addupdate_scatter(deg_v, [col_v[pl.ds(tail_off, L)]], ones,
                           mask=m_tail)

    stage_v[...] = acc
    pltpu.sync_copy(stage_v, shared_mins.at[pl.ds(s_idx * L, L)])
    plsc.subcore_barrier()
    pltpu.sync_copy(shared_mins, allmin_v)
    macc = allmin_v[pl.ds(0, L)]
    for t in range(1, NS):
        macc = jnp.minimum(macc, allmin_v[pl.ds(t * L, L)])
    gmin = jnp.min(macc)

    row_base = c_idx * EC_H

    def cnt_body(i, carry):
        base = i * (L * U)
        for u in range(U):
            off = base + u * L
            cvals = col_v[pl.ds(off, L)]
            rvals = row_v[pl.ds(row_base + off, L)]
            plsc.addupdate_scatter(cnt_v, [rvals], ones, mask=cvals == gmin)
        return carry
    lax.fori_loop(0, H_FULL, cnt_body, 0)
    cvals = col_v[pl.ds(tail_off, L)]
    rvals = row_v[pl.ds(row_base + tail_off, L)]
    plsc.addupdate_scatter(cnt_v, [rvals], ones,
                           mask=m_tail & (cvals == gmin))

    pltpu.sync_copy(deg_v, deg_out.at[wid])
    pltpu.sync_copy(cnt_v, cnt_out.at[wid])

    @pl.when(wid == 0)
    def _():
        stage_v[...] = jnp.full((L,), 0, jnp.int32) + gmin
        pltpu.sync_copy(stage_v, idx_out)


@jax.jit
def _sc_hist(edge_index):
    kern = pl.kernel(
        _sc_body,
        out_type=(
            jax.ShapeDtypeStruct((NW, N), jnp.float32),
            jax.ShapeDtypeStruct((NW, N), jnp.float32),
            jax.ShapeDtypeStruct((L,), jnp.int32),
        ),
        mesh=plsc.VectorSubcoreMesh(core_axis_name="c", subcore_axis_name="s"),
        compiler_params=pltpu.CompilerParams(needs_layout_passes=False),
        scratch_types=[
            pltpu.VMEM((EC_MIN + L,), jnp.int32),
            pltpu.VMEM((EC_H + L,), jnp.int32),
            pltpu.VMEM((N,), jnp.float32),
            pltpu.VMEM((N,), jnp.float32),
            pltpu.VMEM((L,), jnp.int32),
            pltpu.VMEM_SHARED((NS * L,), jnp.int32),
            pltpu.VMEM((NS * L,), jnp.int32),
        ],
    )
    return kern(edge_index)


def _tc_body(idx_ref, degp_ref, cntp_ref, x_ref, w1_ref, wfc_ref,
             bias_ref, bfc_ref, y_ref):
    deg = jnp.sum(degp_ref[...], axis=0, keepdims=True)
    cnt = jnp.sum(cntp_ref[...], axis=0, keepdims=True)
    w = cnt * jnp.sqrt(deg)
    idx = idx_ref[0]
    onehot = (lax.broadcasted_iota(jnp.int32, (1, N), 1) == idx)
    scale = jnp.sqrt(jnp.sum(jnp.where(onehot, deg, 0.0)))
    s = lax.dot_general(w, x_ref[...], (((1,), (0,)), ((), ())),
                        preferred_element_type=jnp.float32,
                        precision=lax.Precision.HIGHEST)
    z = lax.dot_general(s, w1_ref[...], (((1,), (1,)), ((), ())),
                        preferred_element_type=jnp.float32,
                        precision=lax.Precision.HIGHEST)
    out_row = scale * z + bias_ref[...]
    y = lax.dot_general(out_row, wfc_ref[...], (((1,), (1,)), ((), ())),
                        preferred_element_type=jnp.float32,
                        precision=lax.Precision.HIGHEST) + bfc_ref[...]
    y_ref[...] = y


@jax.jit
def _tc_finish(idxv, degp, cntp, x, W1, Wfc, bias2, bfc2):
    return pl.pallas_call(
        _tc_body,
        out_shape=jax.ShapeDtypeStruct((1, O), jnp.float32),
        in_specs=[
            pl.BlockSpec(memory_space=pltpu.SMEM),
            pl.BlockSpec(memory_space=pltpu.VMEM),
            pl.BlockSpec(memory_space=pltpu.VMEM),
            pl.BlockSpec(memory_space=pltpu.VMEM),
            pl.BlockSpec(memory_space=pltpu.VMEM),
            pl.BlockSpec(memory_space=pltpu.VMEM),
            pl.BlockSpec(memory_space=pltpu.VMEM),
            pl.BlockSpec(memory_space=pltpu.VMEM),
        ],
        out_specs=pl.BlockSpec(memory_space=pltpu.VMEM),
    )(idxv, degp, cntp, x, W1, Wfc, bias2, bfc2)


def kernel(x, edge_index, edge_weight, W1, bias, Wfc, bfc):
    ei = jnp.asarray(edge_index, jnp.int32).reshape(2 * E)
    degp, cntp, idxv = _sc_hist(ei)
    y = _tc_finish(idxv, degp, cntp, x, W1, Wfc,
                   bias.reshape(1, H), bfc.reshape(1, O))
    return y.reshape(O)
